# baseline (speedup 1.0000x reference)
import jax
import jax.numpy as jnp
from jax import lax
from jax.experimental import pallas as pl
from jax.experimental.pallas import tpu as pltpu
from jax.experimental.pallas import tpu_sc as plsc

_C, _F, _O, _B = 6, 1000, 256, 16384
_N = _C * _F * _O
_NC, _NS, _L = 2, 16, 16
_NW = _NC * _NS
_BPW = _B // _NW
_ICH = 128
_NL = _BPW // _ICH
_NT = _BPW * 3 // _ICH
_NR = _BPW * 4 // _ICH


def _pose_body(it_hbm, ir_hbm, ot_hbm, oth_hbm, cam_hbm, frm_hbm, obj_hbm,
               otr_hbm, orot_hbm,
               cam_v, frm_v, obj_v, lin_v, lin2_v, idxt_v, idxr_v,
               ta_v, tb_v, q_v, th_v, otr_v, oq_v, sem):
    wid = lax.axis_index("s") * _NC + lax.axis_index("c")
    base = wid * _BPW
    lane = lax.iota(jnp.int32, _L)

    pltpu.sync_copy(cam_hbm.at[pl.ds(base, _BPW)], cam_v)
    pltpu.sync_copy(frm_hbm.at[pl.ds(base, _BPW)], frm_v)
    pltpu.sync_copy(obj_hbm.at[pl.ds(base, _BPW)], obj_v)

    for k in range(_NL):
        def lin_body(j, s, k=k):
            c16 = cam_v[pl.ds(s, _L)]
            f16 = frm_v[pl.ds(s, _L)]
            o16 = obj_v[pl.ds(s, _L)]
            lin = (c16 * _F + f16) * _O + o16
            lin_v[pl.ds(s, _L)] = lin
            lin2_v[k, pl.ds(s - k * _ICH, _L)] = lin
            return s + _L
        lax.fori_loop(0, _ICH // _L, lin_body, k * _ICH)

    three = jnp.full((_L,), 3, jnp.int32)
    for k in range(_NT):
        def idxt_body(j, f, k=k):
            b = lax.div(f, three)
            r = f - b * 3
            idxt_v[k, pl.ds(j * _L, _L)] = plsc.load_gather(lin_v, [b]) * 3 + r
            return f + _L
        lax.fori_loop(0, _ICH // _L, idxt_body, k * _ICH + lane)

    for k in range(_NR):
        def idxr_body(j, f, k=k):
            b = lax.shift_right_logical(f, 2)
            r = lax.bitwise_and(f, 3)
            idxr_v[k, pl.ds(j * _L, _L)] = plsc.load_gather(lin_v, [b]) * 4 + r
            return f + _L
        lax.fori_loop(0, _ICH // _L, idxr_body, k * _ICH + lane)

    copies = []
    for k in range(_NT):
        d = pl.ds(k * _ICH, _ICH)
        copies.append(pltpu.async_copy(it_hbm.at[idxt_v.at[k]], ta_v.at[d], sem))
        copies.append(pltpu.async_copy(ot_hbm.at[idxt_v.at[k]], tb_v.at[d], sem))
    for k in range(_NR):
        d = pl.ds(k * _ICH, _ICH)
        copies.append(pltpu.async_copy(ir_hbm.at[idxr_v.at[k]], q_v.at[d], sem))
    for k in range(_NL):
        d = pl.ds(k * _ICH, _ICH)
        copies.append(pltpu.async_copy(oth_hbm.at[lin2_v.at[k]], th_v.at[d], sem))
    for cp in copies:
        cp.wait()

    def tr_body(j, s):
        otr_v[pl.ds(s, _L)] = ta_v[pl.ds(s, _L)] + tb_v[pl.ds(s, _L)]
        return s + _L
    lax.fori_loop(0, _BPW * 3 // _L, tr_body, 0)

    rev = lane + 3 - 2 * lax.bitwise_and(lane, 3)
    quart = lax.shift_right_logical(lane, 2)
    sgn = jnp.where(lax.bitwise_and(lane, 1) == 1,
                    jnp.float32(1.0), jnp.float32(-1.0))

    def rot_body(j, carry):
        frev, rowq, s16 = carry
        a = q_v[pl.ds(s16, _L)]
        ar = plsc.load_gather(q_v, [frev])
        th = plsc.load_gather(th_v, [rowq])
        h = th * jnp.float32(0.5)
        h2 = h * h
        c = jnp.float32(1.0) + h2 * (
            jnp.float32(-1 / 2) + h2 * (
                jnp.float32(1 / 24) + h2 * (
                    jnp.float32(-1 / 720) + h2 * jnp.float32(1 / 40320))))
        s = h * (jnp.float32(1.0) + h2 * (
            jnp.float32(-1 / 6) + h2 * (
                jnp.float32(1 / 120) + h2 * (
                    jnp.float32(-1 / 5040) + h2 * jnp.float32(1 / 362880)))))
        o = a * c + ar * s * sgn
        oq_v[pl.ds(s16, _L)] = o
        return (frev + _L, rowq + 4, s16 + _L)
    lax.fori_loop(0, _BPW * 4 // _L, rot_body, (rev, quart, 0))

    pltpu.sync_copy(otr_v, otr_hbm.at[pl.ds(base * 3, _BPW * 3)])
    pltpu.sync_copy(oq_v, orot_hbm.at[pl.ds(base * 4, _BPW * 4)])


_pose_call = pl.kernel(
    _pose_body,
    mesh=plsc.VectorSubcoreMesh(core_axis_name="c", subcore_axis_name="s"),
    compiler_params=pltpu.CompilerParams(
        use_tc_tiling_on_sc=False, needs_layout_passes=False),
    out_type=(
        jax.ShapeDtypeStruct((_B * 3,), jnp.float32),
        jax.ShapeDtypeStruct((_B * 4,), jnp.float32),
    ),
    scratch_types=[
        pltpu.VMEM((_BPW,), jnp.int32),
        pltpu.VMEM((_BPW,), jnp.int32),
        pltpu.VMEM((_BPW,), jnp.int32),
        pltpu.VMEM((_BPW,), jnp.int32),
        pltpu.VMEM((_NL, _ICH), jnp.int32),
        pltpu.VMEM((_NT, _ICH), jnp.int32),
        pltpu.VMEM((_NR, _ICH), jnp.int32),
        pltpu.VMEM((_BPW * 3,), jnp.float32),
        pltpu.VMEM((_BPW * 3,), jnp.float32),
        pltpu.VMEM((_BPW * 4,), jnp.float32),
        pltpu.VMEM((_BPW,), jnp.float32),
        pltpu.VMEM((_BPW * 3,), jnp.float32),
        pltpu.VMEM((_BPW * 4,), jnp.float32),
        pltpu.SemaphoreType.DMA,
    ],
)


@jax.jit
def kernel(input_trans, input_rots, opt_trans, opt_rots, cam, frame_idx, obj_id):
    it = input_trans.reshape(-1)
    ir = input_rots.reshape(-1)
    ot = opt_trans.reshape(-1)
    oth = opt_rots.reshape(-1)
    cam = cam.astype(jnp.int32)
    frm = frame_idx.astype(jnp.int32)
    obj = obj_id.astype(jnp.int32)
    tr, rot = _pose_call(it, ir, ot, oth, cam, frm, obj)
    return tr.reshape(_B, 3), rot.reshape(_B, 4)
